# baseline (speedup 1.0000x reference)
import functools

import jax
import jax.numpy as jnp
from jax.experimental import pallas as pl

N = 50000
G = 50
NPG = 1000
NP = 1024
K = 16
NUM_CLASSES = 40
PAD_COORD = 1.0e4
BIG = 1.0e30


def _fused_graph_kernel(posc_ref, posr_ref, w1t_ref, b1_ref, w2t_ref, b2_ref,
                        w3t_ref, b3_ref, w4t_ref, b4_ref, wc_ref, bc_ref,
                        out_ref):
    f32 = jnp.float32
    pc = posc_ref[0]
    pr = posr_ref[0]

    dx = pc[:, 0:1] - pr[0:1, :]
    dy = pc[:, 1:2] - pr[1:2, :]
    d = dx * dx + dy * dy

    rowi = jax.lax.broadcasted_iota(jnp.int32, (NP, NP), 0)
    d = jnp.where(rowi >= NPG, BIG, d)

    w1t = w1t_ref[...]
    u1 = jnp.dot(w1t[:, 0:2] + w1t[:, 2:4], pr, preferred_element_type=f32) \
        + b1_ref[...].reshape(32, 1)
    v1 = -jnp.dot(w1t[:, 2:4], pr, preferred_element_type=f32)
    w2t = w2t_ref[...]
    b2 = b2_ref[...].reshape(32, 1)

    def gather_cols(u, amin):
        offs = jnp.broadcast_to(amin & 127, (32, NP))
        g = None
        for b in range(NP // 128):
            gb = jnp.take_along_axis(u[:, b * 128:(b + 1) * 128], offs,
                                     axis=1, mode="promise_in_bounds")
            g = gb if g is None else jnp.where(amin >> 7 == b, gb, g)
        return g

    idxs = []
    m1 = jnp.full((32, NP), -BIG, dtype=f32)
    for _ in range(K):
        cmin = jnp.min(d, axis=0, keepdims=True)
        is_min = d == cmin
        amin = jnp.min(jnp.where(is_min, rowi, 2 * NP), axis=0, keepdims=True)
        idxs.append(amin)
        d = jnp.where(rowi == amin, BIG, d)
        g1 = gather_cols(u1, amin)
        z = jax.nn.relu(g1 + v1)
        msg = jnp.dot(w2t, z, preferred_element_type=f32) + b2
        m1 = jnp.maximum(m1, msg)

    h1 = jax.nn.relu(m1)

    w3t = w3t_ref[...]
    u2 = (jnp.dot(w3t[:, 0:32], h1, preferred_element_type=f32)
          + jnp.dot(w3t[:, 32:34], pr, preferred_element_type=f32)
          + b3_ref[...].reshape(32, 1))
    v2 = -jnp.dot(w3t[:, 32:34], pr, preferred_element_type=f32)
    w4t = w4t_ref[...]
    b4 = b4_ref[...].reshape(32, 1)

    m2 = jnp.full((32, NP), -BIG, dtype=f32)
    for k in range(K):
        g2 = gather_cols(u2, idxs[k])
        z = jax.nn.relu(g2 + v2)
        msg = jnp.dot(w4t, z, preferred_element_type=f32) + b4
        m2 = jnp.maximum(m2, msg)

    h2 = jax.nn.relu(m2)

    coli = jax.lax.broadcasted_iota(jnp.int32, (32, NP), 1)
    h2 = jnp.where(coli < NPG, h2, -BIG)
    gvec = jnp.max(h2, axis=1).reshape(1, 32)
    logits = jnp.dot(gvec, wc_ref[...], preferred_element_type=f32) + bc_ref[...]
    out = jnp.pad(logits, ((0, 7), (0, 128 - NUM_CLASSES)))
    out_ref[0] = out


@functools.partial(jax.jit, static_argnames=("interpret",))
def _run(pos, W1, b1, W2, b2, W3, b3, W4, b4, Wc, bc, interpret=False):
    pos3 = pos.reshape(G, NPG, 2)
    padc = jnp.full((G, NP - NPG, 2), PAD_COORD, dtype=pos.dtype)
    posc = jnp.concatenate([pos3, padc], axis=1)
    posr = posc.transpose(0, 2, 1)
    full = lambda shape: pl.BlockSpec(shape, lambda g: (0,) * len(shape))
    out = pl.pallas_call(
        _fused_graph_kernel,
        grid=(G,),
        in_specs=[
            pl.BlockSpec((1, NP, 2), lambda g: (g, 0, 0)),
            pl.BlockSpec((1, 2, NP), lambda g: (g, 0, 0)),
            full((32, 4)), full((32,)), full((32, 32)), full((32,)),
            full((32, 34)), full((32,)), full((32, 32)), full((32,)),
            full((32, NUM_CLASSES)), full((NUM_CLASSES,)),
        ],
        out_specs=pl.BlockSpec((1, 8, 128), lambda g: (g, 0, 0)),
        out_shape=jax.ShapeDtypeStruct((G, 8, 128), jnp.float32),
        interpret=interpret,
    )(posc, posr, W1.T, b1, W2.T, b2, W3.T, b3, W4.T, b4, Wc, bc)
    return out[:, 0, :NUM_CLASSES]


def kernel(pos, batch, W1, b1, W2, b2, W3, b3, W4, b4, Wc, bc):
    del batch
    return _run(pos, W1, b1, W2, b2, W3, b3, W4, b4, Wc, bc)


# speedup vs baseline: 1.1017x; 1.1017x over previous
import functools

import jax
import jax.numpy as jnp
from jax.experimental import pallas as pl

N = 50000
G = 50
NPG = 1000
NP = 1024
K = 16
NUM_CLASSES = 40
PG = 2
PAD_COORD = 1.0e4
BIG = 1.0e30


def _fused_graph_kernel(posc_ref, posr_ref, w1t_ref, b1_ref, w2t_ref, b2_ref,
                        w3t_ref, b3_ref, w4t_ref, b4_ref, wc_ref, bc_ref,
                        out_ref):
    f32 = jnp.float32
    w1t = w1t_ref[...]
    w2t = w2t_ref[...]
    b1 = b1_ref[...].reshape(32, 1)
    b2 = b2_ref[...].reshape(32, 1)
    w3t = w3t_ref[...]
    w4t = w4t_ref[...]
    b3 = b3_ref[...].reshape(32, 1)
    b4 = b4_ref[...].reshape(32, 1)
    rowi = jax.lax.broadcasted_iota(jnp.int32, (NP, NP), 0)
    coli = jax.lax.broadcasted_iota(jnp.int32, (32, NP), 1)

    def one_graph(pc, pr):
        dx = pc[:, 0:1] - pr[0:1, :]
        dy = pc[:, 1:2] - pr[1:2, :]
        d = dx * dx + dy * dy
        d = jnp.where(rowi >= NPG, BIG, d)

        u1 = jnp.dot(w1t[:, 0:2] + w1t[:, 2:4], pr,
                     preferred_element_type=f32) + b1
        v1 = -jnp.dot(w1t[:, 2:4], pr, preferred_element_type=f32)

        idxs = []
        m1 = jnp.full((32, NP), -BIG, dtype=f32)
        for _ in range(K):
            cmin = jnp.min(d, axis=0, keepdims=True)
            is_min = d == cmin
            amin = jnp.min(jnp.where(is_min, rowi, 2 * NP), axis=0,
                           keepdims=True)
            idxs.append(amin)
            sel = rowi == amin
            d = jnp.where(sel, BIG, d)
            g1 = jnp.dot(u1, sel.astype(f32), preferred_element_type=f32)
            z = jax.nn.relu(g1 + v1)
            msg = jnp.dot(w2t, z, preferred_element_type=f32) + b2
            m1 = jnp.maximum(m1, msg)

        h1 = jax.nn.relu(m1)

        u2 = (jnp.dot(w3t[:, 0:32], h1, preferred_element_type=f32)
              + jnp.dot(w3t[:, 32:34], pr, preferred_element_type=f32)
              + b3)
        v2 = -jnp.dot(w3t[:, 32:34], pr, preferred_element_type=f32)

        m2 = jnp.full((32, NP), -BIG, dtype=f32)
        for k in range(K):
            sel = (rowi == idxs[k]).astype(f32)
            g2 = jnp.dot(u2, sel, preferred_element_type=f32)
            z = jax.nn.relu(g2 + v2)
            msg = jnp.dot(w4t, z, preferred_element_type=f32) + b4
            m2 = jnp.maximum(m2, msg)

        h2 = jax.nn.relu(m2)

        h2 = jnp.where(coli < NPG, h2, -BIG)
        gvec = jnp.max(h2, axis=1).reshape(1, 32)
        logits = jnp.dot(gvec, wc_ref[...],
                         preferred_element_type=f32) + bc_ref[...]
        return jnp.pad(logits, ((0, 7), (0, 128 - NUM_CLASSES)))

    for t in range(PG):
        out_ref[t] = one_graph(posc_ref[t], posr_ref[t])


@functools.partial(jax.jit, static_argnames=("interpret",))
def _run(pos, W1, b1, W2, b2, W3, b3, W4, b4, Wc, bc, interpret=False):
    pos3 = pos.reshape(G, NPG, 2)
    padc = jnp.full((G, NP - NPG, 2), PAD_COORD, dtype=pos.dtype)
    posc = jnp.concatenate([pos3, padc], axis=1)
    posr = posc.transpose(0, 2, 1)
    full = lambda shape: pl.BlockSpec(shape, lambda g: (0,) * len(shape))
    out = pl.pallas_call(
        _fused_graph_kernel,
        grid=(G // PG,),
        in_specs=[
            pl.BlockSpec((PG, NP, 2), lambda g: (g, 0, 0)),
            pl.BlockSpec((PG, 2, NP), lambda g: (g, 0, 0)),
            full((32, 4)), full((32,)), full((32, 32)), full((32,)),
            full((32, 34)), full((32,)), full((32, 32)), full((32,)),
            full((32, NUM_CLASSES)), full((NUM_CLASSES,)),
        ],
        out_specs=pl.BlockSpec((PG, 8, 128), lambda g: (g, 0, 0)),
        out_shape=jax.ShapeDtypeStruct((G, 8, 128), jnp.float32),
        interpret=interpret,
    )(posc, posr, W1.T, b1, W2.T, b2, W3.T, b3, W4.T, b4, Wc, bc)
    return out[:, 0, :NUM_CLASSES]


def kernel(pos, batch, W1, b1, W2, b2, W3, b3, W4, b4, Wc, bc):
    del batch
    return _run(pos, W1, b1, W2, b2, W3, b3, W4, b4, Wc, bc)


# speedup vs baseline: 1.2178x; 1.1054x over previous
import functools

import jax
import jax.numpy as jnp
from jax.experimental import pallas as pl

N = 50000
G = 50
NPG = 1000
NP = 1024
K = 16
NUM_CLASSES = 40
PAD_COORD = 1.0e4
BIG = 1.0e30


def _fused_graph_kernel(posc_ref, posr_ref, w1t_ref, b1_ref, w2t_ref, b2_ref,
                        w3t_ref, b3_ref, w4t_ref, b4_ref, wc_ref, bc_ref,
                        out_ref):
    f32 = jnp.float32
    pc = posc_ref[0]
    pr = posr_ref[0]

    dx = pc[:, 0:1] - pr[0:1, :]
    dy = pc[:, 1:2] - pr[1:2, :]
    d = dx * dx + dy * dy

    rowi = jax.lax.broadcasted_iota(jnp.int32, (NP, NP), 0)

    w1t = w1t_ref[...]
    a1 = w1t[:, 0:2] + w1t[:, 2:4]
    b1 = b1_ref[...].reshape(32, 1)
    v1 = -jnp.dot(w1t[:, 2:4], pr, preferred_element_type=f32)
    w2t = w2t_ref[...]
    b2 = b2_ref[...].reshape(32, 1)

    idxs = []
    m1 = jnp.full((32, NP), -BIG, dtype=f32)
    for _ in range(K):
        cmin = jnp.min(d, axis=0, keepdims=True)
        is_min = d == cmin
        amin = jnp.min(jnp.where(is_min, rowi, 2 * NP), axis=0, keepdims=True)
        idxs.append(amin)
        sel = rowi == amin
        d = jnp.where(sel, BIG, d)
        pg = jnp.dot(pr, sel.astype(f32), preferred_element_type=f32)
        g1 = jnp.dot(a1, pg, preferred_element_type=f32) + b1
        z = jax.nn.relu(g1 + v1)
        msg = jnp.dot(w2t, z, preferred_element_type=f32) + b2
        m1 = jnp.maximum(m1, msg)

    h1 = jax.nn.relu(m1)

    w3t = w3t_ref[...]
    u2 = (jnp.dot(w3t[:, 0:32], h1, preferred_element_type=f32)
          + jnp.dot(w3t[:, 32:34], pr, preferred_element_type=f32)
          + b3_ref[...].reshape(32, 1))
    v2 = -jnp.dot(w3t[:, 32:34], pr, preferred_element_type=f32)
    w4t = w4t_ref[...]
    b4 = b4_ref[...].reshape(32, 1)

    m2 = jnp.full((32, NP), -BIG, dtype=f32)
    for k in range(K):
        sel = (rowi == idxs[k]).astype(f32)
        g2 = jnp.dot(u2, sel, preferred_element_type=f32)
        z = jax.nn.relu(g2 + v2)
        msg = jnp.dot(w4t, z, preferred_element_type=f32) + b4
        m2 = jnp.maximum(m2, msg)

    h2 = jax.nn.relu(m2)

    coli = jax.lax.broadcasted_iota(jnp.int32, (32, NP), 1)
    h2 = jnp.where(coli < NPG, h2, -BIG)
    gvec = jnp.max(h2, axis=1).reshape(1, 32)
    logits = jnp.dot(gvec, wc_ref[...], preferred_element_type=f32) + bc_ref[...]
    out = jnp.pad(logits, ((0, 7), (0, 128 - NUM_CLASSES)))
    out_ref[0] = out


@functools.partial(jax.jit, static_argnames=("interpret",))
def _run(pos, W1, b1, W2, b2, W3, b3, W4, b4, Wc, bc, interpret=False):
    pos3 = pos.reshape(G, NPG, 2)
    padc = jnp.full((G, NP - NPG, 2), PAD_COORD, dtype=pos.dtype)
    posc = jnp.concatenate([pos3, padc], axis=1)
    posr = posc.transpose(0, 2, 1)
    full = lambda shape: pl.BlockSpec(shape, lambda g: (0,) * len(shape))
    out = pl.pallas_call(
        _fused_graph_kernel,
        grid=(G,),
        in_specs=[
            pl.BlockSpec((1, NP, 2), lambda g: (g, 0, 0)),
            pl.BlockSpec((1, 2, NP), lambda g: (g, 0, 0)),
            full((32, 4)), full((32,)), full((32, 32)), full((32,)),
            full((32, 34)), full((32,)), full((32, 32)), full((32,)),
            full((32, NUM_CLASSES)), full((NUM_CLASSES,)),
        ],
        out_specs=pl.BlockSpec((1, 8, 128), lambda g: (g, 0, 0)),
        out_shape=jax.ShapeDtypeStruct((G, 8, 128), jnp.float32),
        interpret=interpret,
    )(posc, posr, W1.T, b1, W2.T, b2, W3.T, b3, W4.T, b4, Wc, bc)
    return out[:, 0, :NUM_CLASSES]


def kernel(pos, batch, W1, b1, W2, b2, W3, b3, W4, b4, Wc, bc):
    del batch
    return _run(pos, W1, b1, W2, b2, W3, b3, W4, b4, Wc, bc)


# speedup vs baseline: 1.2269x; 1.0075x over previous
import functools

import jax
import jax.numpy as jnp
from jax.experimental import pallas as pl

N = 50000
G = 50
NPG = 1000
NP = 1024
K = 16
NUM_CLASSES = 40
PAD_COORD = 1.0e4
BIG = 1.0e30


def _fused_graph_kernel(posc_ref, posr_ref, w1t_ref, b1_ref, w2t_ref, b2_ref,
                        w3t_ref, b3_ref, w4t_ref, b4_ref, wc_ref, bc_ref,
                        out_ref):
    f32 = jnp.float32
    pc = posc_ref[0]
    pr = posr_ref[0]

    dx = pc[:, 0:1] - pr[0:1, :]
    dy = pc[:, 1:2] - pr[1:2, :]
    d = dx * dx + dy * dy

    rowi = jax.lax.broadcasted_iota(jnp.int32, (NP, NP), 0)

    w1t = w1t_ref[...]
    a1 = w1t[:, 0:2] + w1t[:, 2:4]
    b1 = b1_ref[...].reshape(32, 1)
    v1 = -jnp.dot(w1t[:, 2:4], pr, preferred_element_type=f32)
    w2t = w2t_ref[...]
    b2 = b2_ref[...].reshape(32, 1)

    idxs = []
    m1 = jnp.full((32, NP), -BIG, dtype=f32)
    for k in range(K):
        if k == 0:
            is_min = d == 0.0
        else:
            cmin = jnp.min(d, axis=0, keepdims=True)
            is_min = d == cmin
        amin = jnp.min(jnp.where(is_min, rowi, 2 * NP), axis=0, keepdims=True)
        idxs.append(amin)
        sel = rowi == amin
        d = jnp.where(sel, BIG, d)
        pg = jnp.dot(pr, sel.astype(f32), preferred_element_type=f32)
        g1 = jnp.dot(a1, pg, preferred_element_type=f32) + b1
        z = jax.nn.relu(g1 + v1)
        msg = jnp.dot(w2t, z, preferred_element_type=f32) + b2
        m1 = jnp.maximum(m1, msg)

    h1 = jax.nn.relu(m1)

    w3t = w3t_ref[...]
    u2 = (jnp.dot(w3t[:, 0:32], h1, preferred_element_type=f32)
          + jnp.dot(w3t[:, 32:34], pr, preferred_element_type=f32)
          + b3_ref[...].reshape(32, 1))
    v2 = -jnp.dot(w3t[:, 32:34], pr, preferred_element_type=f32)
    w4t = w4t_ref[...]
    b4 = b4_ref[...].reshape(32, 1)

    m2 = jnp.full((32, NP), -BIG, dtype=f32)
    for k in range(K):
        sel = (rowi == idxs[k]).astype(f32)
        g2 = jnp.dot(u2, sel, preferred_element_type=f32)
        z = jax.nn.relu(g2 + v2)
        msg = jnp.dot(w4t, z, preferred_element_type=f32) + b4
        m2 = jnp.maximum(m2, msg)

    h2 = jax.nn.relu(m2)

    coli = jax.lax.broadcasted_iota(jnp.int32, (32, NP), 1)
    h2 = jnp.where(coli < NPG, h2, -BIG)
    gvec = jnp.max(h2, axis=1).reshape(1, 32)
    logits = jnp.dot(gvec, wc_ref[...], preferred_element_type=f32) + bc_ref[...]
    out = jnp.pad(logits, ((0, 7), (0, 128 - NUM_CLASSES)))
    out_ref[0] = out


@functools.partial(jax.jit, static_argnames=("interpret",))
def _run(pos, W1, b1, W2, b2, W3, b3, W4, b4, Wc, bc, interpret=False):
    pos3 = pos.reshape(G, NPG, 2)
    padc = jnp.full((G, NP - NPG, 2), PAD_COORD, dtype=pos.dtype)
    posc = jnp.concatenate([pos3, padc], axis=1)
    posr = posc.transpose(0, 2, 1)
    full = lambda shape: pl.BlockSpec(shape, lambda g: (0,) * len(shape))
    out = pl.pallas_call(
        _fused_graph_kernel,
        grid=(G,),
        in_specs=[
            pl.BlockSpec((1, NP, 2), lambda g: (g, 0, 0)),
            pl.BlockSpec((1, 2, NP), lambda g: (g, 0, 0)),
            full((32, 4)), full((32,)), full((32, 32)), full((32,)),
            full((32, 34)), full((32,)), full((32, 32)), full((32,)),
            full((32, NUM_CLASSES)), full((NUM_CLASSES,)),
        ],
        out_specs=pl.BlockSpec((1, 8, 128), lambda g: (g, 0, 0)),
        out_shape=jax.ShapeDtypeStruct((G, 8, 128), jnp.float32),
        interpret=interpret,
    )(posc, posr, W1.T, b1, W2.T, b2, W3.T, b3, W4.T, b4, Wc, bc)
    return out[:, 0, :NUM_CLASSES]


def kernel(pos, batch, W1, b1, W2, b2, W3, b3, W4, b4, Wc, bc):
    del batch
    return _run(pos, W1, b1, W2, b2, W3, b3, W4, b4, Wc, bc)
